# baseline (speedup 1.0000x reference)
import functools

import jax
import jax.numpy as jnp
from jax import lax
from jax.experimental import pallas as pl
from jax.experimental.pallas import tpu as pltpu
from jax.experimental.pallas import tpu_sc as plsc

N_NODES = 10000
D = 128
N_EDGES = 320000
BN_EPS = 1e-5

NC = 2
NS = 16
NW = NC * NS
CH = 128
NCH = 80
EPW = NCH * CH
E_PAD = NW * EPW
SLAB = 8
NSLAB = NCH // SLAB
NBUF = 2
PAD_N = 10240
ROWS_PER_TILE = PAD_N // NS


def _sc_aggregate(x, src, dst, zeros):
  mesh = plsc.VectorSubcoreMesh(
      core_axis_name="c", subcore_axis_name="s", num_cores=NC, num_subcores=NS
  )

  @functools.partial(
      pl.kernel,
      out_type=jax.ShapeDtypeStruct((NC, PAD_N, D), jnp.float32),
      mesh=mesh,
      scratch_types=[
          pltpu.VMEM((EPW,), jnp.int32),
          pltpu.VMEM((2 * SLAB, CH), jnp.int32),
          pltpu.VMEM((NBUF, CH, D), jnp.float32),
          pltpu.VMEM_SHARED((PAD_N, D), jnp.float32),
          [pltpu.SemaphoreType.DMA] * NBUF,
          [pltpu.SemaphoreType.DMA] * NBUF,
          [pltpu.SemaphoreType.DMA] * 2,
      ],
  )
  def body(x_hbm, src_hbm, dst_hbm, zeros_hbm, out_hbm,
           src_v, dst_v, rows_v, agg_sh, gsems, ssems, dsems):
    c = lax.axis_index("c")
    s = lax.axis_index("s")
    wid = s * NC + c

    pltpu.sync_copy(zeros_hbm.at[pl.ds(s * ROWS_PER_TILE, ROWS_PER_TILE)],
                    agg_sh.at[pl.ds(s * ROWS_PER_TILE, ROWS_PER_TILE)])
    pltpu.sync_copy(src_hbm.at[wid], src_v)
    plsc.subcore_barrier()

    def refill(sg, p):
      pltpu.async_copy(dst_hbm.at[wid].at[pl.ds(sg * SLAB, SLAB)],
                       dst_v.at[pl.ds(p * SLAB, SLAB)], dsems[p])

    def refill_wait(p):
      pltpu.make_async_copy(dst_hbm.at[0].at[pl.ds(0, SLAB)],
                            dst_v.at[pl.ds(p * SLAB, SLAB)], dsems[p]).wait()

    def gather(k, b):
      pltpu.async_copy(x_hbm.at[src_v.at[pl.ds(k * CH, CH)]], rows_v.at[b],
                       gsems[b])

    def gather_wait(b):
      pltpu.make_async_copy(x_hbm.at[src_v.at[pl.ds(0, CH)]], rows_v.at[b],
                            gsems[b]).wait()

    def scatter(row, b):
      pltpu.async_copy(rows_v.at[b], agg_sh.at[dst_v.at[row]], ssems[b],
                       add=True)

    def scatter_wait(b):
      pltpu.make_async_copy(rows_v.at[b], agg_sh.at[dst_v.at[0]],
                            ssems[b]).wait()

    refill(0, 0)
    refill(1, 1)
    gather(0, 0)
    gather(1, 1)

    def outer(sg2, carry):
      for p in range(2):
        sg = 2 * sg2 + p
        refill_wait(p)
        for t in range(SLAB):
          k = SLAB * sg + t
          b = t % NBUF
          gather_wait(b)
          scatter(p * SLAB + t, b)
          scatter_wait(b)
          gather(jnp.minimum(k + NBUF, NCH - 1), b)
        refill(jnp.minimum(sg + 2, NSLAB - 1), p)
      return carry

    lax.fori_loop(0, NSLAB // 2, outer, 0, unroll=1)
    for b in range(NBUF):
      gather_wait(b)
    for p in range(2):
      refill_wait(p)
    plsc.subcore_barrier()

    pltpu.sync_copy(agg_sh.at[pl.ds(s * ROWS_PER_TILE, ROWS_PER_TILE)],
                    out_hbm.at[c].at[pl.ds(s * ROWS_PER_TILE, ROWS_PER_TILE)])

  return body(x, src.reshape(NW, EPW), dst.reshape(NW, NCH, CH), zeros)


BLK = 400


def _tc_body(x_ref, a0_ref, a1_ref, w1_ref, b1_ref, w2_ref, b2_ref, o_ref):
  xb = x_ref[...]
  h = xb + a0_ref[...] + a1_ref[...]
  h = jnp.maximum(
      jnp.dot(h, w1_ref[...], preferred_element_type=jnp.float32) + b1_ref[...],
      0.0)
  o_ref[...] = (
      jnp.dot(h, w2_ref[...], preferred_element_type=jnp.float32)
      + b2_ref[...] + xb)


def _tc_mlp(x, agg0, agg1, w1t, b1, w2f, b2f):
  grid = (N_NODES // BLK,)
  row_spec = pl.BlockSpec((BLK, D), lambda i: (i, 0))
  full_spec = pl.BlockSpec((D, D), lambda i: (0, 0))
  vec_spec = pl.BlockSpec((1, D), lambda i: (0, 0))
  return pl.pallas_call(
      _tc_body,
      grid=grid,
      in_specs=[row_spec, row_spec, row_spec,
                full_spec, vec_spec, full_spec, vec_spec],
      out_specs=row_spec,
      out_shape=jax.ShapeDtypeStruct((N_NODES, D), jnp.float32),
  )(x, agg0, agg1, w1t, b1.reshape(1, D), w2f, b2f.reshape(1, D))


def kernel(x, edge_index, W1, b1, W2, b2, gamma, beta, running_mean,
           running_var):
  src = edge_index[0].astype(jnp.int32)
  dst = edge_index[1].astype(jnp.int32)
  n_pad = E_PAD - N_EDGES
  src = jnp.concatenate([src, jnp.zeros((n_pad,), jnp.int32)])
  dst = jnp.concatenate([dst, jnp.full((n_pad,), PAD_N - 1, jnp.int32)])
  zeros = jnp.zeros((PAD_N, D), jnp.float32)
  agg = _sc_aggregate(x, src, dst, zeros)

  scale = gamma / jnp.sqrt(running_var + BN_EPS)
  w1t = W1.T
  w2f = W2.T * scale[None, :]
  b2f = b2 * scale + (beta - running_mean * scale)
  return _tc_mlp(x, agg[0, :N_NODES], agg[1, :N_NODES], w1t, b1, w2f, b2f)
